# baseline (speedup 1.0000x reference)
import functools

import jax
import jax.numpy as jnp
from jax import lax
from jax.experimental import pallas as pl
from jax.experimental.pallas import tpu as pltpu
from jax.experimental.pallas import tpu_sc as plsc

VOCAB = 100000
D = 64
N_HYP = 4
BATCH = 1024
SEQ = 200

T = BATCH * SEQ
R = T * N_HYP
NW = 32
ROWS_PER_W = R // NW
GCHUNK = 128
NG = 4
CHUNK = GCHUNK * NG
NCH = ROWS_PER_W // CHUNK
NCHUNKS = R // CHUNK


def _make_sc_kernel():
    mesh = plsc.VectorSubcoreMesh(core_axis_name="c", subcore_axis_name="s")

    @functools.partial(
        pl.kernel,
        mesh=mesh,
        compiler_params=pltpu.CompilerParams(use_tc_tiling_on_sc=False),
        out_type=jax.ShapeDtypeStruct((NCHUNKS, NG, GCHUNK, D), jnp.float32),
        scratch_types=[
            pltpu.VMEM((NG, GCHUNK), jnp.int32),
            pltpu.VMEM((NG, GCHUNK, D), jnp.float32),
            pltpu.VMEM((N_HYP, D), jnp.float32),
            pltpu.SemaphoreType.DMA,
        ],
    )
    def k(tbl_hbm, idx_hbm, coef_hbm, out_hbm, idx_v, rows_v, coef_v, sem):
        nc = 2
        wid = lax.axis_index("s") * nc + lax.axis_index("c")

        pltpu.sync_copy(coef_hbm, coef_v)
        coef_reg = [[coef_v[i, pl.ds(16 * d, 16)] for d in range(D // 16)]
                    for i in range(N_HYP)]

        def chunk_body(c, carry):
            gc = wid * NCH + c
            pltpu.sync_copy(idx_hbm.at[pl.ds(gc * NG, NG)], idx_v)
            copies = [
                pltpu.async_copy(tbl_hbm.at[idx_v.at[j]], rows_v.at[j], sem)
                for j in range(NG)
            ]
            for cp in copies:
                cp.wait()
            for j in range(NG):
                def mul_body(g, _):
                    for i in range(N_HYP):
                        r = 4 * g + i
                        for d in range(D // 16):
                            sl = pl.ds(16 * d, 16)
                            rows_v[j, r, sl] = rows_v[j, r, sl] * coef_reg[i][d]
                    return _
                lax.fori_loop(0, GCHUNK // N_HYP, mul_body, 0)
            pltpu.sync_copy(rows_v, out_hbm.at[gc])
            return carry

        lax.fori_loop(0, NCH, chunk_body, 0)

    return k


_sc_call = _make_sc_kernel()


def kernel(x, tables, phases, amplitudes):
    xf = x.reshape(-1).astype(jnp.int32)
    offs = jnp.arange(N_HYP, dtype=jnp.int32) * VOCAB
    eidx = (xf[:, None] + offs[None, :]).reshape(R // GCHUNK, GCHUNK)
    tflat = tables.reshape(N_HYP * VOCAB, D)
    coef = (jnp.cos(phases) * amplitudes[:, None]).astype(jnp.float32)
    out = _sc_call(tflat, eidx, coef)
    return out.reshape(BATCH, SEQ, N_HYP, D)


# speedup vs baseline: 1.0236x; 1.0236x over previous
import functools

import jax
import jax.numpy as jnp
from jax import lax
from jax.experimental import pallas as pl
from jax.experimental.pallas import tpu as pltpu
from jax.experimental.pallas import tpu_sc as plsc

VOCAB = 100000
D = 64
N_HYP = 4
BATCH = 1024
SEQ = 200

T = BATCH * SEQ
NW = 32
T_PER_W = T // NW
CHUNK = 128
NCH = T_PER_W // CHUNK
NCHUNKS = T // CHUNK
SL = (N_HYP * D) // 128


def _make_sc_kernel():
    mesh = plsc.VectorSubcoreMesh(core_axis_name="c", subcore_axis_name="s")

    @functools.partial(
        pl.kernel,
        mesh=mesh,
        out_type=jax.ShapeDtypeStruct((NCHUNKS, CHUNK, SL, 128), jnp.float32),
        scratch_types=[
            pltpu.VMEM((CHUNK,), jnp.int32),
            pltpu.VMEM((CHUNK, SL, 128), jnp.float32),
            pltpu.VMEM((SL, 128), jnp.float32),
            pltpu.SemaphoreType.DMA,
        ],
    )
    def k(tbl_hbm, idx_hbm, coef_hbm, out_hbm, idx_v, rows_v, coef_v, sem):
        nc = 2
        wid = lax.axis_index("s") * nc + lax.axis_index("c")

        pltpu.sync_copy(coef_hbm, coef_v)
        coef_reg = [[coef_v[s, pl.ds(16 * h, 16)] for h in range(8)]
                    for s in range(SL)]

        def chunk_body(c, carry):
            gc = wid * NCH + c
            pltpu.sync_copy(idx_hbm.at[gc], idx_v)
            pltpu.async_copy(tbl_hbm.at[idx_v], rows_v, sem).wait()

            def mul_body(t, _):
                for s in range(SL):
                    for h in range(8):
                        sl = pl.ds(16 * h, 16)
                        rows_v[t, s, sl] = rows_v[t, s, sl] * coef_reg[s][h]
                return _
            lax.fori_loop(0, CHUNK, mul_body, 0)

            pltpu.sync_copy(rows_v, out_hbm.at[gc])
            return carry

        lax.fori_loop(0, NCH, chunk_body, 0)

    return k


_sc_call = _make_sc_kernel()


def kernel(x, tables, phases, amplitudes):
    xf = x.reshape(NCHUNKS, CHUNK).astype(jnp.int32)
    tbl = tables.transpose(1, 0, 2).reshape(VOCAB, SL, 128)
    coef = (jnp.cos(phases) * amplitudes[:, None]).astype(jnp.float32)
    coef = coef.reshape(SL, 128)
    out = _sc_call(tbl, xf, coef)
    return out.reshape(BATCH, SEQ, N_HYP, D)


# speedup vs baseline: 1.1724x; 1.1453x over previous
import functools

import jax
import jax.numpy as jnp
from jax import lax
from jax.experimental import pallas as pl
from jax.experimental.pallas import tpu as pltpu
from jax.experimental.pallas import tpu_sc as plsc

VOCAB = 100000
D = 64
N_HYP = 4
BATCH = 1024
SEQ = 200

T = BATCH * SEQ
NW = 32
T_PER_W = T // NW
CHUNK = 128
NCH = T_PER_W // CHUNK
NCHUNKS = T // CHUNK
SL = (N_HYP * D) // 128


def _make_sc_kernel():
    mesh = plsc.VectorSubcoreMesh(core_axis_name="c", subcore_axis_name="s")

    @functools.partial(
        pl.kernel,
        mesh=mesh,
        out_type=jax.ShapeDtypeStruct((T, N_HYP * D), jnp.float32),
        scratch_types=[
            pltpu.VMEM((CHUNK,), jnp.int32),
            pltpu.VMEM((CHUNK, N_HYP * D), jnp.float32),
            pltpu.VMEM((N_HYP * D,), jnp.float32),
            pltpu.SemaphoreType.DMA,
        ],
    )
    def k(tbl_hbm, idx_hbm, coef_hbm, out_hbm, idx_v, rows_v, coef_v, sem):
        nc = 2
        wid = lax.axis_index("s") * nc + lax.axis_index("c")

        pltpu.sync_copy(coef_hbm, coef_v)
        coef_reg = [coef_v[pl.ds(16 * h, 16)] for h in range(16)]

        def chunk_body(c, carry):
            gc = wid * NCH + c
            pltpu.sync_copy(idx_hbm.at[gc], idx_v)
            pltpu.async_copy(tbl_hbm.at[idx_v], rows_v, sem).wait()

            def mul_body(t, _):
                for h in range(16):
                    sl = pl.ds(16 * h, 16)
                    rows_v[t, sl] = rows_v[t, sl] * coef_reg[h]
                return _
            lax.fori_loop(0, CHUNK, mul_body, 0)

            pltpu.sync_copy(rows_v, out_hbm.at[pl.ds(gc * CHUNK, CHUNK)])
            return carry

        lax.fori_loop(0, NCH, chunk_body, 0)

    return k


_sc_call = _make_sc_kernel()


def kernel(x, tables, phases, amplitudes):
    xf = x.reshape(NCHUNKS, CHUNK).astype(jnp.int32)
    tbl = tables.transpose(1, 0, 2).reshape(VOCAB, N_HYP * D)
    coef = (jnp.cos(phases) * amplitudes[:, None]).astype(jnp.float32)
    coef = coef.reshape(N_HYP * D)
    out = _sc_call(tbl, xf, coef)
    return out.reshape(BATCH, SEQ, N_HYP, D)
